# baseline (speedup 1.0000x reference)
import functools

import jax
import jax.numpy as jnp
from jax import lax
from jax.experimental import pallas as pl
from jax.experimental.pallas import tpu as pltpu
from jax.experimental.pallas import tpu_sc as plsc

_NC = 2
_NS = 16
_NW = _NC * _NS
_L = 16
_NBUF = 6
_SPLIT = 4


def _masked_copy(s_slabs, n, chunk):
    half = chunk // _SPLIT
    nsub = _SPLIT * s_slabs
    mesh = plsc.VectorSubcoreMesh(core_axis_name="c", subcore_axis_name="s")

    @functools.partial(
        pl.kernel,
        out_type=jax.ShapeDtypeStruct((s_slabs * n,), jnp.float32),
        mesh=mesh,
        scratch_types=[
            pltpu.VMEM((_L,), jnp.int32),
            pltpu.VMEM((half,), jnp.float32),
            pltpu.VMEM_SHARED((half,), jnp.float32),
            [pltpu.VMEM((half,), jnp.float32)] * _NBUF,
            [pltpu.SemaphoreType.DMA] * _NBUF,
            [pltpu.SemaphoreType.DMA] * _NBUF,
            pltpu.SemaphoreType.DMA,
        ],
    )
    def body(x_hbm, m_hbm, out_hbm, m_v, zeros_v, zshared, bufs, gsem, ssem,
             zsem):
        wid = lax.axis_index("s") * _NC + lax.axis_index("c")
        base = wid * chunk

        pltpu.sync_copy(m_hbm, m_v)
        mvec = m_v[...]
        msk = [mvec[s] != 0 for s in range(s_slabs)]
        mint = [mb.astype(jnp.int32) for mb in msk]

        def src_at(i):
            s, h = i // _SPLIT, i % _SPLIT
            return x_hbm.at[pl.ds(s * n + base + h * half, half)]

        def dst_at(i):
            s, h = i // _SPLIT, i % _SPLIT
            return out_hbm.at[pl.ds(s * n + base + h * half, half)]

        for g in range(min(_NBUF - 1, nsub)):
            @pl.when(msk[g // _SPLIT])
            def _pg():
                pltpu.async_copy(src_at(g), bufs[g % _NBUF], gsem[g % _NBUF])

        zvec = jnp.zeros((_L,), jnp.float32)

        def fill(i, _):
            for j in range(16):
                zeros_v[pl.ds((i * 16 + j) * _L, _L)] = zvec
            return 0

        lax.fori_loop(0, half // (_L * 16), fill, 0)

        @pl.when(lax.axis_index("s") == 0)
        def _init_shared():
            pltpu.sync_copy(zeros_v, zshared)

        plsc.subcore_barrier()

        out_cnt = [jnp.int32(0)] * _NBUF

        for idx in range(nsub):
            g = idx + _NBUF - 1
            if g < nsub:
                bg = g % _NBUF
                mg = msk[g // _SPLIT]
                if g >= _NBUF:
                    cond = jnp.logical_and(mg, out_cnt[bg] > 0)

                    @pl.when(cond)
                    def _drain():
                        pltpu.make_async_copy(
                            bufs[bg], dst_at(g), ssem[bg]).wait()

                    out_cnt[bg] = out_cnt[bg] - cond.astype(jnp.int32)

                @pl.when(mg)
                def _gather():
                    pltpu.async_copy(src_at(g), bufs[bg], gsem[bg])

            b = idx % _NBUF
            mi = msk[idx // _SPLIT]

            @pl.when(mi)
            def _copy():
                pltpu.make_async_copy(src_at(idx), bufs[b], gsem[b]).wait()
                pltpu.async_copy(bufs[b], dst_at(idx), ssem[b])

            @pl.when(jnp.logical_not(mi))
            def _zero():
                pltpu.async_copy(zshared, dst_at(idx), zsem)

            out_cnt[b] = out_cnt[b] + mint[idx // _SPLIT]

        for b2 in range(_NBUF):
            @pl.when(out_cnt[b2] > 0)
            def _final_ring_drain():
                pltpu.make_async_copy(
                    bufs[b2], out_hbm.at[pl.ds(base, half)], ssem[b2]).wait()

        msum = mint[0]
        for s in range(1, s_slabs):
            msum = msum + mint[s]
        nzero = jnp.int32(nsub) - _SPLIT * msum

        def zdrain(i, c):
            pltpu.make_async_copy(
                zshared, out_hbm.at[pl.ds(base, half)], zsem).wait()
            return c

        lax.fori_loop(0, nzero, zdrain, 0)

    return body


def kernel(x, mask):
    B, K, H, W, Z = x.shape
    s_slabs = B * K
    n = H * W * Z
    chunk = n // _NW
    xf = x.reshape(s_slabs * n)
    m16 = jnp.zeros((_L,), jnp.int32).at[:s_slabs].set(
        mask.reshape(s_slabs).astype(jnp.int32))
    out = _masked_copy(s_slabs, n, chunk)(xf, m16)
    return out.reshape(B, K, H, W, Z)


# speedup vs baseline: 1.0572x; 1.0572x over previous
import functools

import jax
import jax.numpy as jnp
from jax import lax
from jax.experimental import pallas as pl
from jax.experimental.pallas import tpu as pltpu
from jax.experimental.pallas import tpu_sc as plsc

_NC = 2
_NS = 16
_NW = _NC * _NS
_L = 16
_NBUF = 6
_SPLIT = 4


def _masked_copy(s_slabs, n, chunk):
    half = chunk // _SPLIT
    nsub = _SPLIT * s_slabs
    mesh = plsc.VectorSubcoreMesh(core_axis_name="c", subcore_axis_name="s")

    @functools.partial(
        pl.kernel,
        out_type=jax.ShapeDtypeStruct((s_slabs * n,), jnp.float32),
        mesh=mesh,
        scratch_types=[
            pltpu.VMEM((_L,), jnp.int32),
            pltpu.VMEM((half,), jnp.float32),
            pltpu.VMEM_SHARED((half,), jnp.float32),
            [pltpu.VMEM((half,), jnp.float32)] * _NBUF,
            [pltpu.SemaphoreType.DMA] * _NBUF,
            [pltpu.SemaphoreType.DMA] * _NBUF,
            pltpu.SemaphoreType.DMA,
            pltpu.SemaphoreType.DMA,
        ],
    )
    def body(x_hbm, m_hbm, out_hbm, m_v, zeros_v, zshared, bufs, gsem, ssem,
             zsem, zssem):
        wid = lax.axis_index("s") * _NC + lax.axis_index("c")
        base = wid * chunk

        pltpu.sync_copy(m_hbm, m_v)
        mvec = m_v[...]
        msk = [mvec[s] != 0 for s in range(s_slabs)]
        mint = [mb.astype(jnp.int32) for mb in msk]

        def src_at(i):
            s, h = i // _SPLIT, i % _SPLIT
            return x_hbm.at[pl.ds(s * n + base + h * half, half)]

        def dst_at(i):
            s, h = i // _SPLIT, i % _SPLIT
            return out_hbm.at[pl.ds(s * n + base + h * half, half)]

        for g in range(min(_NBUF - 1, nsub)):
            @pl.when(msk[g // _SPLIT])
            def _pg():
                pltpu.async_copy(src_at(g), bufs[g % _NBUF], gsem[g % _NBUF])

        zvec = jnp.zeros((_L,), jnp.float32)

        def fill(i, _):
            for j in range(16):
                zeros_v[pl.ds((i * 16 + j) * _L, _L)] = zvec
            return 0

        lax.fori_loop(0, half // (_L * 16), fill, 0)

        @pl.when(lax.axis_index("s") == 0)
        def _init_shared():
            pltpu.sync_copy(zeros_v, zshared)

        plsc.subcore_barrier()

        out_cnt = [jnp.int32(0)] * _NBUF

        for idx in range(nsub):
            g = idx + _NBUF - 1
            if g < nsub:
                bg = g % _NBUF
                mg = msk[g // _SPLIT]
                if g >= _NBUF:
                    cond = jnp.logical_and(mg, out_cnt[bg] > 0)

                    @pl.when(cond)
                    def _drain():
                        pltpu.make_async_copy(
                            bufs[bg], dst_at(g), ssem[bg]).wait()

                    out_cnt[bg] = out_cnt[bg] - cond.astype(jnp.int32)

                @pl.when(mg)
                def _gather():
                    pltpu.async_copy(src_at(g), bufs[bg], gsem[bg])

            b = idx % _NBUF
            mi = msk[idx // _SPLIT]

            @pl.when(mi)
            def _copy():
                pltpu.make_async_copy(src_at(idx), bufs[b], gsem[b]).wait()
                pltpu.async_copy(bufs[b], dst_at(idx), ssem[b])

            if idx % _SPLIT < 2:
                @pl.when(jnp.logical_not(mi))
                def _zero_stream():
                    pltpu.async_copy(zeros_v, dst_at(idx), zssem)
            else:
                @pl.when(jnp.logical_not(mi))
                def _zero_dma():
                    pltpu.async_copy(zshared, dst_at(idx), zsem)

            out_cnt[b] = out_cnt[b] + mint[idx // _SPLIT]

        for b2 in range(_NBUF):
            @pl.when(out_cnt[b2] > 0)
            def _final_ring_drain():
                pltpu.make_async_copy(
                    bufs[b2], out_hbm.at[pl.ds(base, half)], ssem[b2]).wait()

        msum = mint[0]
        for s in range(1, s_slabs):
            msum = msum + mint[s]
        nzero_half = 2 * (jnp.int32(s_slabs) - msum)

        def zdrain(i, c):
            pltpu.make_async_copy(
                zshared, out_hbm.at[pl.ds(base, half)], zsem).wait()
            return c

        lax.fori_loop(0, nzero_half, zdrain, 0)

        def zsdrain(i, c):
            pltpu.make_async_copy(
                zeros_v, out_hbm.at[pl.ds(base, half)], zssem).wait()
            return c

        lax.fori_loop(0, nzero_half, zsdrain, 0)

    return body


def kernel(x, mask):
    B, K, H, W, Z = x.shape
    s_slabs = B * K
    n = H * W * Z
    chunk = n // _NW
    xf = x.reshape(s_slabs * n)
    m16 = jnp.zeros((_L,), jnp.int32).at[:s_slabs].set(
        mask.reshape(s_slabs).astype(jnp.int32))
    out = _masked_copy(s_slabs, n, chunk)(xf, m16)
    return out.reshape(B, K, H, W, Z)


# speedup vs baseline: 1.1005x; 1.0410x over previous
import functools

import jax
import jax.numpy as jnp
from jax import lax
from jax.experimental import pallas as pl
from jax.experimental.pallas import tpu as pltpu
from jax.experimental.pallas import tpu_sc as plsc

_NC = 2
_NS = 16
_NW = _NC * _NS
_L = 16
_NBUF = 3
_SPLIT = 2


def _masked_copy(s_slabs, n, chunk):
    half = chunk // _SPLIT
    zhalf = half // 2
    nsub = _SPLIT * s_slabs
    mesh = plsc.VectorSubcoreMesh(core_axis_name="c", subcore_axis_name="s")

    @functools.partial(
        pl.kernel,
        out_type=jax.ShapeDtypeStruct((s_slabs * n,), jnp.float32),
        mesh=mesh,
        scratch_types=[
            pltpu.VMEM((_L,), jnp.int32),
            pltpu.VMEM((zhalf,), jnp.float32),
            [pltpu.VMEM((half,), jnp.float32)] * _NBUF,
            [pltpu.SemaphoreType.DMA] * _NBUF,
            [pltpu.SemaphoreType.DMA] * _NBUF,
        ],
    )
    def body(x_hbm, m_hbm, out_hbm, m_v, zeros_v, bufs, gsem, ssem):
        wid = lax.axis_index("s") * _NC + lax.axis_index("c")
        base = wid * chunk

        pltpu.sync_copy(m_hbm, m_v)
        mvec = m_v[...]

        def src_at(i):
            s, h = i // _SPLIT, i % _SPLIT
            return x_hbm.at[pl.ds(s * n + base + h * half, half)]

        def dst_at(i):
            s, h = i // _SPLIT, i % _SPLIT
            return out_hbm.at[pl.ds(s * n + base + h * half, half)]

        for g in range(min(_NBUF - 1, nsub)):
            @pl.when(mvec[g // _SPLIT] != 0)
            def _pg():
                pltpu.async_copy(src_at(g), bufs[g % _NBUF], gsem[g % _NBUF])

        zvec = jnp.zeros((_L,), jnp.float32)

        def fill(i, _):
            for j in range(16):
                zeros_v[pl.ds((i * 16 + j) * _L, _L)] = zvec
            return 0

        lax.fori_loop(0, zhalf // (_L * 16), fill, 0)

        for idx in range(nsub):
            g = idx + _NBUF - 1
            if g < nsub:
                bg = g % _NBUF
                if g >= _NBUF:
                    pltpu.make_async_copy(bufs[bg], dst_at(g), ssem[bg]).wait()

                @pl.when(mvec[g // _SPLIT] != 0)
                def _gather():
                    pltpu.async_copy(src_at(g), bufs[bg], gsem[bg])

            b = idx % _NBUF
            m = mvec[idx // _SPLIT]

            @pl.when(m != 0)
            def _copy():
                pltpu.make_async_copy(src_at(idx), bufs[b], gsem[b]).wait()
                pltpu.async_copy(bufs[b], dst_at(idx), ssem[b])

            s_i, h_i = idx // _SPLIT, idx % _SPLIT
            off_i = s_i * n + base + h_i * half

            @pl.when(m == 0)
            def _zero():
                pltpu.async_copy(
                    zeros_v, out_hbm.at[pl.ds(off_i, zhalf)], ssem[b])
                pltpu.async_copy(
                    zeros_v, out_hbm.at[pl.ds(off_i + zhalf, zhalf)], ssem[b])

        for idx in range(nsub - _NBUF, nsub):
            pltpu.make_async_copy(
                bufs[idx % _NBUF], dst_at(idx), ssem[idx % _NBUF]).wait()

    return body


def kernel(x, mask):
    B, K, H, W, Z = x.shape
    s_slabs = B * K
    n = H * W * Z
    chunk = n // _NW
    xf = x.reshape(s_slabs * n)
    m16 = jnp.zeros((_L,), jnp.int32).at[:s_slabs].set(
        mask.reshape(s_slabs).astype(jnp.int32))
    out = _masked_copy(s_slabs, n, chunk)(xf, m16)
    return out.reshape(B, K, H, W, Z)
